# baseline (speedup 1.0000x reference)
import functools

import jax
import jax.numpy as jnp
from jax import lax
from jax.experimental import pallas as pl
from jax.experimental.pallas import tpu as pltpu
from jax.experimental.pallas import tpu_sc as plsc

E = 8
K = 2
D = 768
L = 2048
H = 4 * D

A = K * L
TB = 128
G = (A + E * (TB - 1)) // TB + 1
P = G * TB
GPAD = 64

NW = 32
RPW = P // NW
RC = 32
TPW = L // NW
CB = 16


def _gelu_exact(h):
    return 0.5 * h * (1.0 + lax.erf(h * (2.0 ** -0.5)))


def _excl_cumsum_cols(m, chunk=256):
    n = m.shape[0]
    ri = lax.broadcasted_iota(jnp.int32, (chunk, chunk), 0)
    ci = lax.broadcasted_iota(jnp.int32, (chunk, chunk), 1)
    tstrict = (ci < ri).astype(jnp.float32)
    carry = jnp.zeros((1, m.shape[1]), jnp.float32)
    parts = []
    for c in range(n // chunk):
        blk = m[c * chunk:(c + 1) * chunk, :]
        parts.append(jnp.dot(tstrict, blk, preferred_element_type=jnp.float32)
                     + carry)
        carry = carry + jnp.sum(blk, axis=0, keepdims=True)
    return jnp.concatenate(parts, axis=0)


def _router_body(x_ref, wg_ref, pos_ref, g_ref, be_ref, loss_ref):
    x = x_ref[...]
    wg = wg_ref[...]
    logits = jnp.dot(x, wg, preferred_element_type=jnp.float32)
    lane = lax.broadcasted_iota(jnp.int32, (L, E), 1)
    m1 = jnp.max(logits, axis=1, keepdims=True)
    i1 = jnp.min(jnp.where(logits == m1, lane, E), axis=1, keepdims=True)
    oh1 = (lane == i1).astype(jnp.float32)
    logits2 = jnp.where(lane == i1, -jnp.inf, logits)
    m2 = jnp.max(logits2, axis=1, keepdims=True)
    i2 = jnp.min(jnp.where(logits2 == m2, lane, E), axis=1, keepdims=True)
    oh2 = (lane == i2).astype(jnp.float32)
    a = jnp.exp(m2 - m1)
    g1 = 1.0 / (1.0 + a)
    g2 = a / (1.0 + a)
    g_ref[...] = jnp.concatenate([g1, g2], axis=1)

    counts = jnp.sum(oh1 + oh2, axis=0, keepdims=True)
    cn = counts / A
    loss_ref[...] = jnp.sum((cn - 1.0 / E) ** 2, axis=1, keepdims=True) / E

    pc = jnp.floor((counts + (TB - 1)) / TB) * TB
    ei = lax.broadcasted_iota(jnp.int32, (E, E), 0)
    ej = lax.broadcasted_iota(jnp.int32, (E, E), 1)
    te = (ei < ej).astype(jnp.float32)
    off = jnp.dot(pc, te, preferred_element_type=jnp.float32)
    end = off + pc

    cs = _excl_cumsum_cols(oh1 + oh2)
    p0 = jnp.sum(oh1 * (off + cs), axis=1, keepdims=True)
    p1 = jnp.sum(oh2 * (off + cs + oh1), axis=1, keepdims=True)
    pos_ref[...] = jnp.concatenate([p0, p1], axis=1).astype(jnp.int32)

    b128 = lax.broadcasted_iota(jnp.int32, (GPAD, E), 0).astype(jnp.float32) * TB
    bc = jnp.sum((b128 >= end).astype(jnp.float32), axis=1, keepdims=True)
    be_ref[...] = jnp.minimum(bc, E - 1).astype(jnp.int32)


def _dispatch_body(pos_hbm, g_hbm, x_hbm, xs_hbm, sg_hbm,
                   pos_v, g_v, st_v, sg_v, rows_v, sem):
    wid = lax.axis_index("s") * 2 + lax.axis_index("c")
    pltpu.sync_copy(pos_hbm, pos_v)
    pltpu.sync_copy(g_hbm, g_v)

    zi = jnp.zeros((16,), jnp.int32)
    zf = jnp.zeros((16,), jnp.float32)

    def init_body(i, carry):
        st_v[pl.ds(i * 16, 16)] = zi
        sg_v[pl.ds(i * 16, 16)] = zf
        return carry

    lax.fori_loop(0, P // 16, init_body, 0)

    iota16 = lax.iota(jnp.int32, 16)

    def scat_body(i, carry):
        idx = pos_v[pl.ds(i * 16, 16)]
        tok = lax.shift_right_logical(i * 16 + iota16, 1)
        plsc.store_scatter(st_v, [idx], tok)
        plsc.store_scatter(sg_v, [idx], g_v[pl.ds(i * 16, 16)])
        return carry

    lax.fori_loop(0, A // 16, scat_body, 0)

    def gather_body(c, carry):
        base = wid * RPW + c * RC
        pltpu.async_copy(x_hbm.at[st_v.at[pl.ds(base, RC)]], rows_v,
                         sem).wait()
        pltpu.sync_copy(rows_v, xs_hbm.at[pl.ds(base, RC)])
        return carry

    lax.fori_loop(0, RPW // RC, gather_body, 0)

    @pl.when(wid == 0)
    def _():
        pltpu.sync_copy(sg_v, sg_hbm)


def _gmm_body(be_ref, xs_ref, w1_ref, w2_ref, sg_ref, ys_ref):
    xb = xs_ref[...]
    h = _gelu_exact(jnp.dot(xb, w1_ref[0], preferred_element_type=jnp.float32))
    y = jnp.dot(h, w2_ref[0], preferred_element_type=jnp.float32)
    ys_ref[...] = y * sg_ref[...]


def _combine_body(ys_hbm, i0_hbm, i1_hbm, out_hbm,
                  i0_v, i1_v, a_v, b_v, o_v, sem0, sem1):
    wid = lax.axis_index("s") * 2 + lax.axis_index("c")
    base_t = wid * TPW
    pltpu.sync_copy(i0_hbm.at[pl.ds(base_t, TPW)], i0_v)
    pltpu.sync_copy(i1_hbm.at[pl.ds(base_t, TPW)], i1_v)

    def chunk_body(c, carry):
        cp0 = pltpu.async_copy(ys_hbm.at[i0_v.at[pl.ds(c * CB, CB)]],
                               a_v, sem0)
        cp1 = pltpu.async_copy(ys_hbm.at[i1_v.at[pl.ds(c * CB, CB)]],
                               b_v, sem1)
        cp0.wait()
        cp1.wait()

        def row_body(r, rc):
            for k in range(D // 16):
                s = pl.ds(k * 16, 16)
                o_v[r, s] = a_v[r, s] + b_v[r, s]
            return rc

        lax.fori_loop(0, CB, row_body, 0)
        pltpu.sync_copy(o_v, out_hbm.at[pl.ds(base_t + c * CB, CB)])
        return carry

    lax.fori_loop(0, TPW // CB, chunk_body, 0)


def _make_sc_kernels():
    mesh = plsc.VectorSubcoreMesh(core_axis_name="c", subcore_axis_name="s",
                                  num_cores=2, num_subcores=16)
    sc_params = pltpu.CompilerParams(needs_layout_passes=False)
    dispatch = pl.kernel(
        _dispatch_body,
        compiler_params=sc_params,
        out_type=(
            jax.ShapeDtypeStruct((P, D), jnp.float32),
            jax.ShapeDtypeStruct((P,), jnp.float32),
        ),
        mesh=mesh,
        scratch_types=[
            pltpu.VMEM((A,), jnp.int32),
            pltpu.VMEM((A,), jnp.float32),
            pltpu.VMEM((P,), jnp.int32),
            pltpu.VMEM((P,), jnp.float32),
            pltpu.VMEM((RC, D), jnp.float32),
            pltpu.SemaphoreType.DMA,
        ],
    )
    combine = pl.kernel(
        _combine_body,
        compiler_params=sc_params,
        out_type=jax.ShapeDtypeStruct((L, D), jnp.float32),
        mesh=mesh,
        scratch_types=[
            pltpu.VMEM((TPW,), jnp.int32),
            pltpu.VMEM((TPW,), jnp.int32),
            pltpu.VMEM((CB, D), jnp.float32),
            pltpu.VMEM((CB, D), jnp.float32),
            pltpu.VMEM((CB, D), jnp.float32),
            pltpu.SemaphoreType.DMA,
            pltpu.SemaphoreType.DMA,
        ],
    )
    return dispatch, combine


def kernel(x, Wg, W1, W2):
    _dispatch, _combine = _make_sc_kernels()
    x2 = x.reshape(L, D)

    pos, g01, be, loss = pl.pallas_call(
        _router_body,
        out_shape=(
            jax.ShapeDtypeStruct((L, K), jnp.int32),
            jax.ShapeDtypeStruct((L, K), jnp.float32),
            jax.ShapeDtypeStruct((GPAD, 1), jnp.int32),
            jax.ShapeDtypeStruct((1, 1), jnp.float32),
        ),
    )(x2, Wg)

    xs, sg = _dispatch(pos.reshape(A), g01.reshape(A), x2)

    grid_spec = pltpu.PrefetchScalarGridSpec(
        num_scalar_prefetch=1,
        grid=(G,),
        in_specs=[
            pl.BlockSpec((TB, D), lambda b, be: (b, 0)),
            pl.BlockSpec((1, D, H), lambda b, be: (be[b], 0, 0)),
            pl.BlockSpec((1, H, D), lambda b, be: (be[b], 0, 0)),
            pl.BlockSpec((TB, 1), lambda b, be: (b, 0)),
        ],
        out_specs=pl.BlockSpec((TB, D), lambda b, be: (b, 0)),
    )
    ys = pl.pallas_call(
        _gmm_body,
        grid_spec=grid_spec,
        out_shape=jax.ShapeDtypeStruct((P, D), jnp.float32),
    )(be.reshape(GPAD)[:G], xs, W1, W2, sg.reshape(P, 1))

    out = _combine(ys, pos[:, 0], pos[:, 1])

    return out.reshape(1, L, D), loss.reshape(())
